# baseline (speedup 1.0000x reference)
import functools

import jax
import jax.numpy as jnp
from jax import lax
from jax.experimental import pallas as pl
from jax.experimental.pallas import tpu as pltpu
from jax.experimental.pallas import tpu_sc as plsc

N = 10000
E = 160000
F = 256
HALF = 128
NS = 16
CHUNK = 32
EDGES_PER_TILE = 10240
NCHUNK = EDGES_PER_TILE // CHUNK
EP = EDGES_PER_TILE * NS
JUNK_ROW = N
ACC_ROWS = 10112
ZROWS = ACC_ROWS // NS
CP = 80
NCP = N // CP
CP_ITERS = -(-NCP // NS)

RB = 2000
NRB = N // RB


def _mm1_body(x_ref, w_ref, o_ref):
    o_ref[...] = jax.lax.dot_general(
        x_ref[...], w_ref[...], (((1,), (0,)), ((), ())),
        precision=jax.lax.Precision.HIGHEST,
        preferred_element_type=jnp.float32)


def _tc_matmul1(x, W1):
    return pl.pallas_call(
        _mm1_body,
        grid=(NRB, 2),
        in_specs=[pl.BlockSpec((RB, F), lambda i, j: (i, 0)),
                  pl.BlockSpec((F, HALF), lambda i, j: (0, j))],
        out_specs=pl.BlockSpec((RB, HALF), lambda i, j: (j * NRB + i, 0)),
        out_shape=jax.ShapeDtypeStruct((2 * N, HALF), jnp.float32),
    )(x, W1)


def _mm2_body(a0_ref, a1_ref, w_ref, o_ref):
    a0 = jnp.maximum(a0_ref[...], 0.0)
    a1 = jnp.maximum(a1_ref[...], 0.0)
    dot = functools.partial(
        jax.lax.dot_general,
        dimension_numbers=(((1,), (0,)), ((), ())),
        precision=jax.lax.Precision.HIGHEST,
        preferred_element_type=jnp.float32)
    o_ref[...] = dot(a0, w_ref[:HALF, :]) + dot(a1, w_ref[HALF:, :])


def _tc_matmul2(agg, W2):
    return pl.pallas_call(
        _mm2_body,
        grid=(NRB, 2),
        in_specs=[pl.BlockSpec((RB, HALF), lambda i, j: (i, 0)),
                  pl.BlockSpec((RB, HALF), lambda i, j: (NRB + i, 0)),
                  pl.BlockSpec((F, HALF), lambda i, j: (0, j))],
        out_specs=pl.BlockSpec((RB, HALF), lambda i, j: (j * NRB + i, 0)),
        out_shape=jax.ShapeDtypeStruct((2 * N, HALF), jnp.float32),
    )(agg, agg, W2)


NBUF = 6
KA = NBUF - 1
CPR = 128 // CHUNK
IDXR = NCHUNK // CPR
NG = NCHUNK // NBUF
NTAIL = NCHUNK - NG * NBUF


def _make_sc_aggregate(strided_out: bool):
    mesh = plsc.VectorSubcoreMesh(core_axis_name="c", subcore_axis_name="s")
    out_shape = (N, F) if strided_out else (2 * N, HALF)

    @functools.partial(
        pl.kernel,
        out_type=jax.ShapeDtypeStruct(out_shape, jnp.float32),
        mesh=mesh,
        scratch_types=[
            pltpu.VMEM((IDXR, 128), jnp.int32),
            pltpu.VMEM((IDXR, 128), jnp.int32),
            pltpu.VMEM_SHARED((ACC_ROWS, HALF), jnp.float32),
        ] + [pltpu.VMEM((CHUNK, HALF), jnp.float32)] * NBUF
          + [pltpu.SemaphoreType.DMA] * (2 * NBUF),
    )
    def agg(h_hbm, src_hbm, dst_hbm, z_hbm, o_hbm, sidx, didx, acc, *bufs):
        rows = bufs[:NBUF]
        gsem = bufs[NBUF:2 * NBUF]
        ssem = bufs[2 * NBUF:]
        c = lax.axis_index("c")
        s = lax.axis_index("s")

        z0 = pltpu.async_copy(z_hbm.at[pl.ds(s * ZROWS, ZROWS)],
                              acc.at[pl.ds(s * ZROWS, ZROWS)], gsem[0])
        z1 = pltpu.async_copy(src_hbm.at[pl.ds((c * NS + s) * IDXR, IDXR)],
                              sidx, gsem[1])
        z2 = pltpu.async_copy(dst_hbm.at[pl.ds(s * IDXR, IDXR)], didx,
                              gsem[2])
        z0.wait()
        z1.wait()
        z2.wait()
        plsc.subcore_barrier()

        def gather_desc(b, cc):
            idx = sidx.at[cc // CPR, pl.ds((cc % CPR) * CHUNK, CHUNK)]
            return pltpu.make_async_copy(h_hbm.at[idx], rows[b], gsem[b])

        def scatter_desc(b, cc):
            idx = didx.at[cc // CPR, pl.ds((cc % CPR) * CHUNK, CHUNK)]
            return pltpu.make_async_copy(rows[b], acc.at[idx], ssem[b])

        for b in range(KA):
            gather_desc(b, b).start()

        @pl.loop(0, NG)
        def _(g):
            for b in range(NBUF):
                cc = g * NBUF + b
                gather_desc(b, cc).wait()
                scatter_desc(b, cc).start(add=True)

                def _drain(b=b, cc=cc):
                    scatter_desc((b - 1) % NBUF, cc - 1).wait()

                if b >= 1:
                    _drain()
                else:
                    pl.when(g >= 1)(_drain)

                @pl.when(cc + KA < NCHUNK)
                def _():
                    gather_desc((b + KA) % NBUF, cc + KA).start()

        for t in range(NTAIL):
            cc = NG * NBUF + t
            gather_desc(cc % NBUF, cc).wait()
            scatter_desc(cc % NBUF, cc).start(add=True)
            scatter_desc((cc - 1) % NBUF, cc - 1).wait()
        scatter_desc((NCHUNK - 1) % NBUF, NCHUNK - 1).wait()
        plsc.subcore_barrier()

        def out_desc(q):
            r0 = q * CP
            if strided_out:
                return pltpu.make_async_copy(
                    acc.at[pl.ds(r0, CP)],
                    o_hbm.at[pl.ds(r0, CP), pl.ds(c * HALF, HALF)], gsem[0])
            return pltpu.make_async_copy(
                acc.at[pl.ds(r0, CP)], o_hbm.at[pl.ds(c * N + r0, CP)],
                gsem[0])

        @pl.loop(0, CP_ITERS)
        def _(j):
            q = s + NS * j

            @pl.when(q < NCP)
            def _():
                out_desc(q).start()

        @pl.loop(0, CP_ITERS)
        def _(j):
            q = s + NS * j

            @pl.when(q < NCP)
            def _():
                out_desc(q).wait()

    return agg


_sc_agg_mid = _make_sc_aggregate(strided_out=False)
_sc_agg_out = _make_sc_aggregate(strided_out=True)


def kernel(x, edge_index, W1, W2):
    ei = edge_index.astype(jnp.int32)
    src = ei[0]
    dst = ei[1]
    pad = EP - E
    src_p = jnp.concatenate([src, jnp.zeros((pad,), jnp.int32)])
    dst_p = jnp.concatenate([dst, jnp.full((pad,), JUNK_ROW, jnp.int32)])
    src2 = jnp.concatenate([src_p, src_p + N]).reshape(2 * NS * IDXR, 128)
    dst_p = dst_p.reshape(NS * IDXR, 128)
    zeros_acc = jnp.zeros((ACC_ROWS, HALF), jnp.float32)

    h1 = _tc_matmul1(x.astype(jnp.float32), W1)
    agg1 = _sc_agg_mid(h1, src2, dst_p, zeros_acc)
    h2 = _tc_matmul2(agg1, W2)
    out = _sc_agg_out(h2, src2, dst_p, zeros_acc)
    return out


# speedup vs baseline: 1.0134x; 1.0134x over previous
import functools

import jax
import jax.numpy as jnp
from jax import lax
from jax.experimental import pallas as pl
from jax.experimental.pallas import tpu as pltpu
from jax.experimental.pallas import tpu_sc as plsc

N = 10000
E = 160000
F = 256
HALF = 128
NS = 16
CHUNK = 32
EDGES_PER_TILE = 10240
NCHUNK = EDGES_PER_TILE // CHUNK
EP = EDGES_PER_TILE * NS
JUNK_ROW = N
ACC_ROWS = 10112
ZROWS = ACC_ROWS // NS
CP = 80
NCP = N // CP
CP_ITERS = -(-NCP // NS)

RB = 2000
NRB = N // RB


def _mm1_body(x_ref, w_ref, o_ref):
    o_ref[...] = jax.lax.dot_general(
        x_ref[...], w_ref[...], (((1,), (0,)), ((), ())),
        precision=jax.lax.Precision.HIGHEST,
        preferred_element_type=jnp.float32)


def _tc_matmul1(x, W1):
    return pl.pallas_call(
        _mm1_body,
        grid=(NRB, 2),
        in_specs=[pl.BlockSpec((RB, F), lambda i, j: (i, 0)),
                  pl.BlockSpec((F, HALF), lambda i, j: (0, j))],
        out_specs=pl.BlockSpec((RB, HALF), lambda i, j: (j * NRB + i, 0)),
        out_shape=jax.ShapeDtypeStruct((2 * N, HALF), jnp.float32),
    )(x, W1)


def _mm2_body(a0_ref, a1_ref, w_ref, o_ref):
    a0 = jnp.maximum(a0_ref[...], 0.0)
    a1 = jnp.maximum(a1_ref[...], 0.0)
    dot = functools.partial(
        jax.lax.dot_general,
        dimension_numbers=(((1,), (0,)), ((), ())),
        precision=jax.lax.Precision.HIGHEST,
        preferred_element_type=jnp.float32)
    o_ref[...] = dot(a0, w_ref[:HALF, :]) + dot(a1, w_ref[HALF:, :])


def _tc_matmul2(agg, W2):
    return pl.pallas_call(
        _mm2_body,
        grid=(NRB, 2),
        in_specs=[pl.BlockSpec((RB, HALF), lambda i, j: (i, 0)),
                  pl.BlockSpec((RB, HALF), lambda i, j: (NRB + i, 0)),
                  pl.BlockSpec((F, HALF), lambda i, j: (0, j))],
        out_specs=pl.BlockSpec((RB, HALF), lambda i, j: (j * NRB + i, 0)),
        out_shape=jax.ShapeDtypeStruct((2 * N, HALF), jnp.float32),
    )(agg, agg, W2)


NBUF = 7
KA = NBUF - 1
CPR = 128 // CHUNK
IDXR = NCHUNK // CPR
NG = NCHUNK // NBUF
NTAIL = NCHUNK - NG * NBUF


def _make_sc_aggregate(strided_out: bool):
    mesh = plsc.VectorSubcoreMesh(core_axis_name="c", subcore_axis_name="s")
    out_shape = (N, F) if strided_out else (2 * N, HALF)

    @functools.partial(
        pl.kernel,
        out_type=jax.ShapeDtypeStruct(out_shape, jnp.float32),
        mesh=mesh,
        scratch_types=[
            pltpu.VMEM((IDXR, 128), jnp.int32),
            pltpu.VMEM((IDXR, 128), jnp.int32),
            pltpu.VMEM_SHARED((ACC_ROWS, HALF), jnp.float32),
        ] + [pltpu.VMEM((CHUNK, HALF), jnp.float32)] * NBUF
          + [pltpu.SemaphoreType.DMA] * (2 * NBUF),
    )
    def agg(h_hbm, src_hbm, dst_hbm, z_hbm, o_hbm, sidx, didx, acc, *bufs):
        rows = bufs[:NBUF]
        gsem = bufs[NBUF:2 * NBUF]
        ssem = bufs[2 * NBUF:]
        c = lax.axis_index("c")
        s = lax.axis_index("s")

        z0 = pltpu.async_copy(z_hbm.at[pl.ds(s * ZROWS, ZROWS)],
                              acc.at[pl.ds(s * ZROWS, ZROWS)], gsem[0])
        z1 = pltpu.async_copy(src_hbm.at[pl.ds((c * NS + s) * IDXR, IDXR)],
                              sidx, gsem[1])
        z2 = pltpu.async_copy(dst_hbm.at[pl.ds(s * IDXR, IDXR)], didx,
                              gsem[2])
        z0.wait()
        z1.wait()
        z2.wait()
        plsc.subcore_barrier()

        def gather_desc(b, cc):
            idx = sidx.at[cc // CPR, pl.ds((cc % CPR) * CHUNK, CHUNK)]
            return pltpu.make_async_copy(h_hbm.at[idx], rows[b], gsem[b])

        def scatter_desc(b, cc):
            idx = didx.at[cc // CPR, pl.ds((cc % CPR) * CHUNK, CHUNK)]
            return pltpu.make_async_copy(rows[b], acc.at[idx], ssem[b])

        for b in range(KA):
            gather_desc(b, b).start()

        @pl.loop(0, NG)
        def _(g):
            for b in range(NBUF):
                cc = g * NBUF + b
                gather_desc(b, cc).wait()
                scatter_desc(b, cc).start(add=True)

                def _drain(b=b, cc=cc):
                    scatter_desc((b - 1) % NBUF, cc - 1).wait()

                if b >= 1:
                    _drain()
                else:
                    pl.when(g >= 1)(_drain)

                @pl.when(cc + KA < NCHUNK)
                def _():
                    gather_desc((b + KA) % NBUF, cc + KA).start()

        for t in range(NTAIL):
            cc = NG * NBUF + t
            gather_desc(cc % NBUF, cc).wait()
            scatter_desc(cc % NBUF, cc).start(add=True)
            scatter_desc((cc - 1) % NBUF, cc - 1).wait()
        scatter_desc((NCHUNK - 1) % NBUF, NCHUNK - 1).wait()
        plsc.subcore_barrier()

        def out_desc(q):
            r0 = q * CP
            if strided_out:
                return pltpu.make_async_copy(
                    acc.at[pl.ds(r0, CP)],
                    o_hbm.at[pl.ds(r0, CP), pl.ds(c * HALF, HALF)], gsem[0])
            return pltpu.make_async_copy(
                acc.at[pl.ds(r0, CP)], o_hbm.at[pl.ds(c * N + r0, CP)],
                gsem[0])

        @pl.loop(0, CP_ITERS)
        def _(j):
            q = s + NS * j

            @pl.when(q < NCP)
            def _():
                out_desc(q).start()

        @pl.loop(0, CP_ITERS)
        def _(j):
            q = s + NS * j

            @pl.when(q < NCP)
            def _():
                out_desc(q).wait()

    return agg


_sc_agg_mid = _make_sc_aggregate(strided_out=False)
_sc_agg_out = _make_sc_aggregate(strided_out=True)


def kernel(x, edge_index, W1, W2):
    ei = edge_index.astype(jnp.int32)
    src = ei[0]
    dst = ei[1]
    pad = EP - E
    src_p = jnp.concatenate([src, jnp.zeros((pad,), jnp.int32)])
    dst_p = jnp.concatenate([dst, jnp.full((pad,), JUNK_ROW, jnp.int32)])
    src2 = jnp.concatenate([src_p, src_p + N]).reshape(2 * NS * IDXR, 128)
    dst_p = dst_p.reshape(NS * IDXR, 128)
    zeros_acc = jnp.zeros((ACC_ROWS, HALF), jnp.float32)

    h1 = _tc_matmul1(x.astype(jnp.float32), W1)
    agg1 = _sc_agg_mid(h1, src2, dst_p, zeros_acc)
    h2 = _tc_matmul2(agg1, W2)
    out = _sc_agg_out(h2, src2, dst_p, zeros_acc)
    return out
